# baseline (speedup 1.0000x reference)
import functools

import jax
import jax.numpy as jnp
from jax import lax
from jax.experimental import pallas as pl
from jax.experimental.pallas import tpu as pltpu
from jax.experimental.pallas import tpu_sc as plsc

N = 10000
NP = 10240
E = 320000
DIN = 128
HID = 16
NCLS = 47
NC, NS = 2, 16
NW = NC * NS
SS = 2048
NQ = 5
EPW = NQ * SS
EPAD = NW * EPW
STRIPE = NP // NS
RB = 256


def _make_agg_body(spmem_tab):
    def body(tab_hbm, src_hbm, dst_hbm, zeros_hbm, out_hbm, *scratch):
        if spmem_tab:
            (acc_a, acc_b, tab_sh, srcv, dstv, gbuf, obuf, gsem,
             ssem) = scratch
        else:
            (acc_a, acc_b, srcv, dstv, gbuf, obuf, gsem, ssem) = scratch
        cid = lax.axis_index("c")
        sid = lax.axis_index("s")
        wid = cid * NS + sid
        pltpu.sync_copy(zeros_hbm, obuf)
        pltpu.sync_copy(obuf, acc_a.at[pl.ds(sid * STRIPE, STRIPE)])
        pltpu.sync_copy(obuf, acc_b.at[pl.ds(sid * STRIPE, STRIPE)])
        if spmem_tab:
            pltpu.sync_copy(tab_hbm.at[pl.ds(sid * STRIPE, STRIPE)], obuf)
            pltpu.sync_copy(obuf, tab_sh.at[pl.ds(sid * STRIPE, STRIPE)])
            tab = tab_sh
        else:
            tab = tab_hbm
        pltpu.sync_copy(src_hbm.at[wid], srcv)
        pltpu.sync_copy(dst_hbm.at[wid], dstv)
        plsc.subcore_barrier()

        def fire_g(q, h):
            pltpu.async_copy(tab.at[srcv.at[q]], gbuf.at[h], gsem)

        def fire_s(q, h):
            @pl.when(sid % 2 == 0)
            def _():
                pltpu.async_copy(gbuf.at[h], acc_a.at[dstv.at[q]],
                                 ssem, add=True)

            @pl.when(sid % 2 == 1)
            def _():
                pltpu.async_copy(gbuf.at[h], acc_b.at[dstv.at[q]],
                                 ssem, add=True)

        def drain_g(q, h):
            pltpu.make_async_copy(tab_hbm.at[srcv.at[0]],
                                  gbuf.at[h], gsem).wait()

        def drain_s(q, h):
            pltpu.make_async_copy(gbuf.at[h],
                                  acc_a.at[dstv.at[0]], ssem).wait()

        fire_g(0, 0)
        for q in range(NQ):
            h = q % 2
            drain_g(q, h)
            if q >= 1:
                drain_s(q - 1, 1 - h)
            fire_s(q, h)
            if q + 1 < NQ:
                fire_g(q + 1, 1 - h)
        drain_s(NQ - 1, (NQ - 1) % 2)
        plsc.subcore_barrier()
        pltpu.sync_copy(acc_a.at[pl.ds(sid * STRIPE, STRIPE)], obuf)
        pltpu.sync_copy(obuf, out_hbm.at[cid, 0, pl.ds(sid * STRIPE, STRIPE)])
        pltpu.sync_copy(acc_b.at[pl.ds(sid * STRIPE, STRIPE)], obuf)
        pltpu.sync_copy(obuf, out_hbm.at[cid, 1, pl.ds(sid * STRIPE, STRIPE)])

    return body


@functools.cache
def _sc_agg(spmem_tab):
    shared = [pltpu.VMEM_SHARED((NP, HID), jnp.float32),
              pltpu.VMEM_SHARED((NP, HID), jnp.float32)]
    if spmem_tab:
        shared.append(pltpu.VMEM_SHARED((NP, HID), jnp.float32))
    return pl.kernel(
        _make_agg_body(spmem_tab),
        out_type=jax.ShapeDtypeStruct((NC, 2, NP, HID), jnp.float32),
        mesh=plsc.VectorSubcoreMesh(core_axis_name="c", subcore_axis_name="s"),
        scratch_types=shared + [
            pltpu.VMEM((NQ, SS), jnp.int32),
            pltpu.VMEM((NQ, SS), jnp.int32),
            pltpu.VMEM((2, SS, HID), jnp.float32),
            pltpu.VMEM((STRIPE, HID), jnp.float32),
            pltpu.SemaphoreType.DMA,
            pltpu.SemaphoreType.DMA,
        ],
        compiler_params=pltpu.CompilerParams(use_tc_tiling_on_sc=False),
    )


def _sc_deg_body(ones_hbm, dst_hbm, zeros_hbm, out_hbm,
                 acc_sh, dstv, ones_v, obuf, ssem):
    cid = lax.axis_index("c")
    sid = lax.axis_index("s")
    wid = cid * NS + sid
    pltpu.sync_copy(zeros_hbm.at[pl.ds(sid * STRIPE, STRIPE)], obuf)
    pltpu.sync_copy(obuf, acc_sh.at[pl.ds(sid * STRIPE, STRIPE)])
    pltpu.sync_copy(dst_hbm.at[wid], dstv)
    pltpu.sync_copy(ones_hbm, ones_v)
    plsc.subcore_barrier()

    for q in range(NQ):
        pltpu.async_copy(ones_v, acc_sh.at[dstv.at[q]], ssem, add=True)
    for q in range(NQ):
        pltpu.make_async_copy(ones_v, acc_sh.at[dstv.at[0]], ssem).wait()
    plsc.subcore_barrier()
    pltpu.sync_copy(acc_sh.at[pl.ds(sid * STRIPE, STRIPE)], obuf)
    pltpu.sync_copy(obuf, out_hbm.at[cid, pl.ds(sid * STRIPE, STRIPE)])


@functools.cache
def _sc_deg():
    return pl.kernel(
        _sc_deg_body,
        out_type=jax.ShapeDtypeStruct((NC, NP, HID), jnp.float32),
        mesh=plsc.VectorSubcoreMesh(core_axis_name="c", subcore_axis_name="s"),
        scratch_types=[
            pltpu.VMEM_SHARED((NP, HID), jnp.float32),
            pltpu.VMEM((NQ, SS), jnp.int32),
            pltpu.VMEM((SS, HID), jnp.float32),
            pltpu.VMEM((STRIPE, HID), jnp.float32),
            pltpu.SemaphoreType.DMA,
        ],
        compiler_params=pltpu.CompilerParams(use_tc_tiling_on_sc=False),
    )


def _tc_g0_body(x_ref, w_ref, degp_ref, dis_ref, g0_ref):
    deg = degp_ref[0] + degp_ref[1]
    dis = lax.rsqrt(deg + 1.0)
    dis_ref[...] = dis
    h = jnp.dot(x_ref[...], w_ref[...], preferred_element_type=jnp.float32)
    g0_ref[...] = h * dis


def _tc_mid_body(accp_ref, g_ref, dis_ref, jk_ref, w_ref, b_ref,
                 gout_ref, jkout_ref):
    acc = ((accp_ref[0] + accp_ref[1]) + (accp_ref[2] + accp_ref[3]))
    dis = dis_ref[...]
    h = jnp.maximum(dis * (acc + g_ref[...]) + b_ref[...], 0.0)
    jk = jnp.maximum(jk_ref[...], h)
    jkout_ref[...] = jk
    gout_ref[...] = jnp.dot(h, w_ref[...],
                            preferred_element_type=jnp.float32) * dis


def _tc_fin_body(accp_ref, g_ref, dis_ref, jk_ref, b_ref, fcw_ref, fcb_ref,
                 out_ref):
    acc = ((accp_ref[0] + accp_ref[1]) + (accp_ref[2] + accp_ref[3]))
    dis = dis_ref[...]
    h = jnp.maximum(dis * (acc + g_ref[...]) + b_ref[...], 0.0)
    jk = jnp.maximum(jk_ref[...], h)
    logits = jnp.dot(jk, fcw_ref[...],
                     preferred_element_type=jnp.float32) + fcb_ref[...]
    m = jnp.max(logits, axis=1, keepdims=True)
    lse = jnp.log(jnp.sum(jnp.exp(logits - m), axis=1, keepdims=True))
    out_ref[...] = logits - m - lse


_GRID = (NP // RB,)
_row = lambda i: (i, 0)
_whole = lambda i: (0, 0)
_part = lambda i: (0, i, 0)

_tc_g0 = pl.pallas_call(
    _tc_g0_body,
    grid=_GRID,
    in_specs=[
        pl.BlockSpec((RB, DIN), _row),
        pl.BlockSpec((DIN, HID), _whole),
        pl.BlockSpec((NC, RB, HID), _part),
    ],
    out_specs=[pl.BlockSpec((RB, HID), _row)] * 2,
    out_shape=[jax.ShapeDtypeStruct((NP, HID), jnp.float32)] * 2,
)

_tc_mid = pl.pallas_call(
    _tc_mid_body,
    grid=_GRID,
    in_specs=[
        pl.BlockSpec((4, RB, HID), _part),
        pl.BlockSpec((RB, HID), _row),
        pl.BlockSpec((RB, HID), _row),
        pl.BlockSpec((RB, HID), _row),
        pl.BlockSpec((HID, HID), _whole),
        pl.BlockSpec((1, HID), _whole),
    ],
    out_specs=[pl.BlockSpec((RB, HID), _row)] * 2,
    out_shape=[jax.ShapeDtypeStruct((NP, HID), jnp.float32)] * 2,
)

_tc_fin = pl.pallas_call(
    _tc_fin_body,
    grid=_GRID,
    in_specs=[
        pl.BlockSpec((4, RB, HID), _part),
        pl.BlockSpec((RB, HID), _row),
        pl.BlockSpec((RB, HID), _row),
        pl.BlockSpec((RB, HID), _row),
        pl.BlockSpec((1, HID), _whole),
        pl.BlockSpec((HID, NCLS), _whole),
        pl.BlockSpec((1, NCLS), _whole),
    ],
    out_specs=pl.BlockSpec((RB, NCLS), _row),
    out_shape=jax.ShapeDtypeStruct((NP, NCLS), jnp.float32),
)


def kernel(x, edge_index, W0, b0, W1, b1, W2, b2, W3, b3, W4, b4, W5, b5,
           fcW, fcb):
    xp = jnp.pad(x, ((0, NP - N), (0, 0)))
    pad_rows = (N + (jnp.arange(EPAD - E, dtype=jnp.int32) % (NP - N)))
    src_a = jnp.concatenate([edge_index[0], pad_rows]).reshape(NW, NQ, SS)
    dst_a = jnp.concatenate([edge_index[1], pad_rows]).reshape(NW, NQ, SS)
    zeros = jnp.zeros((NP, HID), jnp.float32)
    zstripe = jnp.zeros((STRIPE, HID), jnp.float32)
    ones = jnp.ones((SS, HID), jnp.float32)

    degp = _sc_deg()(ones, dst_a, zeros)
    dis, g = _tc_g0(xp, W0, degp)
    jk = zeros
    Ws = [W1, W2, W3, W4, W5]
    bs = [b0, b1, b2, b3, b4]
    for i in range(5):
        accp = _sc_agg(True)(g, src_a, dst_a, zstripe).reshape(4, NP, HID)
        g, jk = _tc_mid(accp, g, dis, jk, Ws[i], bs[i].reshape(1, HID))
    accp = _sc_agg(True)(g, src_a, dst_a, zstripe).reshape(4, NP, HID)
    out = _tc_fin(accp, g, dis, jk, b5.reshape(1, HID), fcW,
                  fcb.reshape(1, NCLS))
    return out[:N]


# speedup vs baseline: 1.1194x; 1.1194x over previous
import functools

import jax
import jax.numpy as jnp
from jax import lax
from jax.experimental import pallas as pl
from jax.experimental.pallas import tpu as pltpu
from jax.experimental.pallas import tpu_sc as plsc

N = 10000
NP = 10240
E = 320000
DIN = 128
HID = 16
NCLS = 47
NC, NS = 2, 16
NW = NC * NS
SS = 2048
NQ = 5
EPW = NQ * SS
EPAD = NW * EPW
STRIPE = NP // NS
RB = 256


def _make_agg_body(spmem_tab):
    def body(tab_hbm, src_hbm, dst_hbm, zeros_hbm, out_hbm, *scratch):
        if spmem_tab:
            (acc_sh, tab_sh, srcv, dstv, gbuf, obuf, gsem, ssem) = scratch
        else:
            (acc_sh, srcv, dstv, gbuf, obuf, gsem, ssem) = scratch
        cid = lax.axis_index("c")
        sid = lax.axis_index("s")
        wid = cid * NS + sid
        pltpu.sync_copy(zeros_hbm, obuf)
        pltpu.sync_copy(obuf, acc_sh.at[pl.ds(sid * STRIPE, STRIPE)])
        if spmem_tab:
            pltpu.sync_copy(tab_hbm.at[pl.ds(sid * STRIPE, STRIPE)], obuf)
            pltpu.sync_copy(obuf, tab_sh.at[pl.ds(sid * STRIPE, STRIPE)])
            tab = tab_sh
        else:
            tab = tab_hbm
        pltpu.sync_copy(src_hbm.at[wid], srcv)
        pltpu.sync_copy(dst_hbm.at[wid], dstv)
        plsc.subcore_barrier()

        def fire_g(q, h):
            pltpu.async_copy(tab.at[srcv.at[q]], gbuf.at[h], gsem)

        def fire_s(q, h):
            pltpu.async_copy(gbuf.at[h], acc_sh.at[dstv.at[q]],
                             ssem, add=True)

        def drain_g(q, h):
            pltpu.make_async_copy(tab_hbm.at[srcv.at[0]],
                                  gbuf.at[h], gsem).wait()

        def drain_s(q, h):
            pltpu.make_async_copy(gbuf.at[h],
                                  acc_sh.at[dstv.at[0]], ssem).wait()

        fire_g(0, 0)
        for q in range(NQ):
            h = q % 2
            drain_g(q, h)
            if q >= 1:
                drain_s(q - 1, 1 - h)
            fire_s(q, h)
            if q + 1 < NQ:
                fire_g(q + 1, 1 - h)
        drain_s(NQ - 1, (NQ - 1) % 2)
        plsc.subcore_barrier()
        pltpu.sync_copy(acc_sh.at[pl.ds(sid * STRIPE, STRIPE)], obuf)
        pltpu.sync_copy(obuf, out_hbm.at[cid, pl.ds(sid * STRIPE, STRIPE)])

    return body


@functools.cache
def _sc_agg(spmem_tab):
    shared = [pltpu.VMEM_SHARED((NP, HID), jnp.float32)]
    if spmem_tab:
        shared.append(pltpu.VMEM_SHARED((NP, HID), jnp.float32))
    return pl.kernel(
        _make_agg_body(spmem_tab),
        out_type=jax.ShapeDtypeStruct((NC, NP, HID), jnp.float32),
        mesh=plsc.VectorSubcoreMesh(core_axis_name="c", subcore_axis_name="s"),
        scratch_types=shared + [
            pltpu.VMEM((NQ, SS), jnp.int32),
            pltpu.VMEM((NQ, SS), jnp.int32),
            pltpu.VMEM((2, SS, HID), jnp.float32),
            pltpu.VMEM((STRIPE, HID), jnp.float32),
            pltpu.SemaphoreType.DMA,
            pltpu.SemaphoreType.DMA,
        ],
        compiler_params=pltpu.CompilerParams(use_tc_tiling_on_sc=False),
    )


def _sc_deg_body(ones_hbm, dst_hbm, zeros_hbm, out_hbm,
                 acc_sh, dstv, ones_v, obuf, ssem):
    cid = lax.axis_index("c")
    sid = lax.axis_index("s")
    wid = cid * NS + sid
    pltpu.sync_copy(zeros_hbm.at[pl.ds(sid * STRIPE, STRIPE)], obuf)
    pltpu.sync_copy(obuf, acc_sh.at[pl.ds(sid * STRIPE, STRIPE)])
    pltpu.sync_copy(dst_hbm.at[wid], dstv)
    pltpu.sync_copy(ones_hbm, ones_v)
    plsc.subcore_barrier()

    for q in range(NQ):
        pltpu.async_copy(ones_v, acc_sh.at[dstv.at[q]], ssem, add=True)
    for q in range(NQ):
        pltpu.make_async_copy(ones_v, acc_sh.at[dstv.at[0]], ssem).wait()
    plsc.subcore_barrier()
    pltpu.sync_copy(acc_sh.at[pl.ds(sid * STRIPE, STRIPE)], obuf)
    pltpu.sync_copy(obuf, out_hbm.at[cid, pl.ds(sid * STRIPE, STRIPE)])


@functools.cache
def _sc_deg():
    return pl.kernel(
        _sc_deg_body,
        out_type=jax.ShapeDtypeStruct((NC, NP, HID), jnp.float32),
        mesh=plsc.VectorSubcoreMesh(core_axis_name="c", subcore_axis_name="s"),
        scratch_types=[
            pltpu.VMEM_SHARED((NP, HID), jnp.float32),
            pltpu.VMEM((NQ, SS), jnp.int32),
            pltpu.VMEM((SS, HID), jnp.float32),
            pltpu.VMEM((STRIPE, HID), jnp.float32),
            pltpu.SemaphoreType.DMA,
        ],
        compiler_params=pltpu.CompilerParams(use_tc_tiling_on_sc=False),
    )


def _tc_g0_body(x_ref, w_ref, degp_ref, dis_ref, g0_ref):
    deg = degp_ref[0] + degp_ref[1]
    dis = lax.rsqrt(deg + 1.0)
    dis_ref[...] = dis
    h = jnp.dot(x_ref[...], w_ref[...], preferred_element_type=jnp.float32)
    g0_ref[...] = h * dis


def _tc_mid_body(accp_ref, g_ref, dis_ref, jk_ref, w_ref, b_ref,
                 gout_ref, jkout_ref):
    acc = accp_ref[0] + accp_ref[1]
    dis = dis_ref[...]
    h = jnp.maximum(dis * (acc + g_ref[...]) + b_ref[...], 0.0)
    jk = jnp.maximum(jk_ref[...], h)
    jkout_ref[...] = jk
    gout_ref[...] = jnp.dot(h, w_ref[...],
                            preferred_element_type=jnp.float32) * dis


def _tc_fin_body(accp_ref, g_ref, dis_ref, jk_ref, b_ref, fcw_ref, fcb_ref,
                 out_ref):
    acc = accp_ref[0] + accp_ref[1]
    dis = dis_ref[...]
    h = jnp.maximum(dis * (acc + g_ref[...]) + b_ref[...], 0.0)
    jk = jnp.maximum(jk_ref[...], h)
    logits = jnp.dot(jk, fcw_ref[...],
                     preferred_element_type=jnp.float32) + fcb_ref[...]
    m = jnp.max(logits, axis=1, keepdims=True)
    lse = jnp.log(jnp.sum(jnp.exp(logits - m), axis=1, keepdims=True))
    out_ref[...] = logits - m - lse


_GRID = (NP // RB,)
_row = lambda i: (i, 0)
_whole = lambda i: (0, 0)
_part = lambda i: (0, i, 0)

_tc_g0 = pl.pallas_call(
    _tc_g0_body,
    grid=_GRID,
    in_specs=[
        pl.BlockSpec((RB, DIN), _row),
        pl.BlockSpec((DIN, HID), _whole),
        pl.BlockSpec((NC, RB, HID), _part),
    ],
    out_specs=[pl.BlockSpec((RB, HID), _row)] * 2,
    out_shape=[jax.ShapeDtypeStruct((NP, HID), jnp.float32)] * 2,
)

_tc_mid = pl.pallas_call(
    _tc_mid_body,
    grid=_GRID,
    in_specs=[
        pl.BlockSpec((NC, RB, HID), _part),
        pl.BlockSpec((RB, HID), _row),
        pl.BlockSpec((RB, HID), _row),
        pl.BlockSpec((RB, HID), _row),
        pl.BlockSpec((HID, HID), _whole),
        pl.BlockSpec((1, HID), _whole),
    ],
    out_specs=[pl.BlockSpec((RB, HID), _row)] * 2,
    out_shape=[jax.ShapeDtypeStruct((NP, HID), jnp.float32)] * 2,
)

_tc_fin = pl.pallas_call(
    _tc_fin_body,
    grid=_GRID,
    in_specs=[
        pl.BlockSpec((NC, RB, HID), _part),
        pl.BlockSpec((RB, HID), _row),
        pl.BlockSpec((RB, HID), _row),
        pl.BlockSpec((RB, HID), _row),
        pl.BlockSpec((1, HID), _whole),
        pl.BlockSpec((HID, NCLS), _whole),
        pl.BlockSpec((1, NCLS), _whole),
    ],
    out_specs=pl.BlockSpec((RB, NCLS), _row),
    out_shape=jax.ShapeDtypeStruct((NP, NCLS), jnp.float32),
)


def kernel(x, edge_index, W0, b0, W1, b1, W2, b2, W3, b3, W4, b4, W5, b5,
           fcW, fcb):
    xp = jnp.pad(x, ((0, NP - N), (0, 0)))
    pad_rows = (N + (jnp.arange(EPAD - E, dtype=jnp.int32) % (NP - N)))
    src_a = jnp.concatenate([edge_index[0], pad_rows]).reshape(NW, NQ, SS)
    dst_a = jnp.concatenate([edge_index[1], pad_rows]).reshape(NW, NQ, SS)
    zeros = jnp.zeros((NP, HID), jnp.float32)
    zstripe = jnp.zeros((STRIPE, HID), jnp.float32)
    ones = jnp.ones((SS, HID), jnp.float32)

    degp = _sc_deg()(ones, dst_a, zeros)
    dis, g = _tc_g0(xp, W0, degp)
    jk = zeros
    Ws = [W1, W2, W3, W4, W5]
    bs = [b0, b1, b2, b3, b4]
    for i in range(5):
        accp = _sc_agg(True)(g, src_a, dst_a, zstripe)
        g, jk = _tc_mid(accp, g, dis, jk, Ws[i], bs[i].reshape(1, HID))
    accp = _sc_agg(True)(g, src_a, dst_a, zstripe)
    out = _tc_fin(accp, g, dis, jk, b5.reshape(1, HID), fcW,
                  fcb.reshape(1, NCLS))
    return out[:N]
